# Initial kernel scaffold; baseline (speedup 1.0000x reference)
import functools

import jax
import jax.numpy as jnp
from jax import lax
from jax.experimental import pallas as pl
from jax.experimental.pallas import tpu as pltpu
from jax.experimental.pallas import tpu_sc as plsc

N = 10000
D = 128
G = 128
OUT = 2
E = 320000

NC = 2
NS = 16
NW = NC * NS

CHUNK = 128
CPW = -(-E // (NW * CHUNK))
EPW = CPW * CHUNK
EPAD = EPW * NW
ACC_ROWS = 10112
RPS = 632
LAST_INIT = N - (NS - 1) * RPS

RP = 320
RP_LAST = N - (NW - 1) * RP
GAD = G * D

_mesh = plsc.VectorSubcoreMesh(core_axis_name="c", subcore_axis_name="s")


@functools.partial(
    pl.kernel,
    out_type=jax.ShapeDtypeStruct((NC, N, D), jnp.float32),
    mesh=_mesh,
    scratch_types=[
        pltpu.VMEM((CHUNK,), jnp.int32),
        pltpu.VMEM((CHUNK,), jnp.int32),
        pltpu.VMEM((CHUNK, D), jnp.float32),
        pltpu.VMEM_SHARED((ACC_ROWS, D), jnp.float32),
        pltpu.SemaphoreType.DMA,
    ],
)
def _sc_agg(h_hbm, z_hbm, src_hbm, dst_hbm, out_hbm, src_v, dst_v, rows_v,
            acc_sh, sem):
    c = lax.axis_index("c")
    s = lax.axis_index("s")
    wid = s * NC + c

    def init_from(ref):
        @pl.when(s < NS - 1)
        def _():
            pltpu.sync_copy(ref.at[pl.ds(s * RPS, RPS)],
                            acc_sh.at[pl.ds(s * RPS, RPS)])

        @pl.when(s == NS - 1)
        def _():
            pltpu.sync_copy(ref.at[pl.ds((NS - 1) * RPS, LAST_INIT)],
                            acc_sh.at[pl.ds((NS - 1) * RPS, LAST_INIT)])

    @pl.when(c == 0)
    def _():
        init_from(h_hbm)

    @pl.when(c == 1)
    def _():
        init_from(z_hbm)

    plsc.subcore_barrier()

    def chunk_body(i, carry):
        off = wid * EPW + i * CHUNK
        pltpu.sync_copy(src_hbm.at[pl.ds(off, CHUNK)], src_v)
        pltpu.sync_copy(dst_hbm.at[pl.ds(off, CHUNK)], dst_v)
        pltpu.async_copy(h_hbm.at[src_v], rows_v, sem).wait()
        pltpu.sync_copy(rows_v, acc_sh.at[dst_v], add=True)
        return carry

    lax.fori_loop(0, CPW, chunk_body, 0)
    plsc.subcore_barrier()

    @pl.when(s < NS - 1)
    def _():
        pltpu.sync_copy(acc_sh.at[pl.ds(s * RPS, RPS)],
                        out_hbm.at[c, pl.ds(s * RPS, RPS)])

    @pl.when(s == NS - 1)
    def _():
        pltpu.sync_copy(acc_sh.at[pl.ds((NS - 1) * RPS, LAST_INIT)],
                        out_hbm.at[c, pl.ds((NS - 1) * RPS, LAST_INIT)])


@functools.partial(
    pl.kernel,
    out_type=jax.ShapeDtypeStruct((NW, GAD), jnp.float32),
    mesh=_mesh,
    scratch_types=[
        pltpu.VMEM((RP * D,), jnp.float32),
        pltpu.VMEM((RP,), jnp.int32),
        pltpu.VMEM((GAD,), jnp.float32),
    ],
)
def _sc_pool(hf_hbm, b_hbm, ninf_hbm, out_hbm, rows_v, bid_v, acc_v):
    c = lax.axis_index("c")
    s = lax.axis_index("s")
    wid = s * NC + c

    pltpu.sync_copy(ninf_hbm, acc_v)

    @pl.when(wid < NW - 1)
    def _():
        pltpu.sync_copy(hf_hbm.at[pl.ds(wid * RP * D, RP * D)], rows_v)
        pltpu.sync_copy(b_hbm.at[pl.ds(wid * RP, RP)], bid_v)

    @pl.when(wid == NW - 1)
    def _():
        pltpu.sync_copy(hf_hbm.at[pl.ds((NW - 1) * RP * D, RP_LAST * D)],
                        rows_v.at[pl.ds(0, RP_LAST * D)])
        pltpu.sync_copy(b_hbm.at[pl.ds((NW - 1) * RP, RP_LAST)],
                        bid_v.at[pl.ds(0, RP_LAST)])

    nrows = jnp.where(wid == NW - 1, RP_LAST, RP)

    def row_body(r, carry):
        g = bid_v[r]
        gbase = g * D
        rbase = r * D
        for v in range(D // 16):
            xv = rows_v[pl.ds(rbase + v * 16, 16)]
            cv = acc_v[pl.ds(gbase + v * 16, 16)]
            acc_v[pl.ds(gbase + v * 16, 16)] = jnp.maximum(cv, xv)
        return carry

    lax.fori_loop(0, nrows, row_body, 0)
    pltpu.sync_copy(acc_v, out_hbm.at[wid])


def _mlp_body(parts_ref, w1_ref, b1_ref, g1_ref, be1_ref, w2_ref, b2_ref,
              bng_ref, bnb_ref, out_ref):
    z = parts_ref[0] + parts_ref[1]
    z = jnp.dot(z, w1_ref[...], preferred_element_type=jnp.float32,
                precision=lax.Precision.HIGHEST) + b1_ref[...]
    m = jnp.mean(z, axis=0)
    v = jnp.mean((z - m) ** 2, axis=0)
    z = (z - m) / jnp.sqrt(v + 1e-5) * g1_ref[...] + be1_ref[...]
    z = jnp.maximum(z, 0.0)
    z = jnp.dot(z, w2_ref[...], preferred_element_type=jnp.float32,
                precision=lax.Precision.HIGHEST) + b2_ref[...]
    m2 = jnp.mean(z, axis=0)
    v2 = jnp.mean((z - m2) ** 2, axis=0)
    z = (z - m2) / jnp.sqrt(v2 + 1e-5) * bng_ref[...] + bnb_ref[...]
    out_ref[...] = jnp.maximum(z, 0.0)


_tc_mlp = pl.pallas_call(
    _mlp_body,
    out_shape=jax.ShapeDtypeStruct((N, D), jnp.float32),
)


def _head_body(p0_ref, p1_ref, p2_ref, p3_ref,
               w10, b10, w20, b20, w11, b11, w21, b21,
               w12, b12, w22, b22, w13, b13, w23, b23, out_ref):
    score = jnp.zeros((G, OUT), jnp.float32)
    heads = [(p0_ref, w10, b10, w20, b20),
             (p1_ref, w11, b11, w21, b21),
             (p2_ref, w12, b12, w22, b22),
             (p3_ref, w13, b13, w23, b23)]
    for pref, w1, b1, w2, b2 in heads:
        arr = pref[...]
        pooled = arr[0:G, :]
        for w in range(1, NW):
            pooled = jnp.maximum(pooled, arr[w * G:(w + 1) * G, :])
        o = jnp.dot(pooled, w1[...], preferred_element_type=jnp.float32,
                    precision=lax.Precision.HIGHEST) + b1[...]
        o = jnp.maximum(o, 0.0)
        o = jnp.dot(o, w2[...], preferred_element_type=jnp.float32,
                    precision=lax.Precision.HIGHEST) + b2[...]
        score = score + o
    out_ref[...] = score


_tc_head = pl.pallas_call(
    _head_body,
    out_shape=jax.ShapeDtypeStruct((G, OUT), jnp.float32),
)


def kernel(x, edge_index, batch, params):
    src = edge_index[0]
    dst = edge_index[1]
    npad = EPAD - E
    srcp = jnp.concatenate([src, jnp.zeros((npad,), jnp.int32)])
    dstp = jnp.concatenate([dst, jnp.full((npad,), N, jnp.int32)])
    zeros_nd = jnp.zeros((N, D), jnp.float32)
    ninf = jnp.full((GAD,), -jnp.inf, jnp.float32)

    h = x
    hiddens = [x]
    for i in range(3):
        p = params["conv%d" % i]
        parts = _sc_agg(h, zeros_nd, srcp, dstp)
        h = _tc_mlp(parts, p["W1"], p["b1"], p["g1"], p["be1"],
                    p["W2"], p["b2"], p["bn_g"], p["bn_b"])
        hiddens.append(h)

    pooled_parts = [
        _sc_pool(hh.reshape(N * D), batch, ninf).reshape(NW * G, D)
        for hh in hiddens
    ]

    hp = [params["head%d" % i] for i in range(4)]
    score = _tc_head(
        pooled_parts[0], pooled_parts[1], pooled_parts[2], pooled_parts[3],
        hp[0]["W1"], hp[0]["b1"], hp[0]["W2"], hp[0]["b2"],
        hp[1]["W1"], hp[1]["b1"], hp[1]["W2"], hp[1]["b2"],
        hp[2]["W1"], hp[2]["b1"], hp[2]["W2"], hp[2]["b2"],
        hp[3]["W1"], hp[3]["b1"], hp[3]["W2"], hp[3]["b2"],
    )
    return score


# speedup vs baseline: 3.3739x; 3.3739x over previous
import functools

import jax
import jax.numpy as jnp
from jax import lax
from jax.experimental import pallas as pl
from jax.experimental.pallas import tpu as pltpu
from jax.experimental.pallas import tpu_sc as plsc

N = 10000
D = 128
G = 128
OUT = 2
E = 320000

NC = 2
NS = 16
NW = NC * NS

CHUNK = 128
CPW = -(-E // (NW * CHUNK))
EPW = CPW * CHUNK
EPAD = EPW * NW
ACC_ROWS = 10112
RPS = 632
LAST_INIT = N - (NS - 1) * RPS

RP = 320
RP_LAST = N - (NW - 1) * RP
GAD = G * D

_mesh = plsc.VectorSubcoreMesh(core_axis_name="c", subcore_axis_name="s")


@functools.partial(
    pl.kernel,
    out_type=jax.ShapeDtypeStruct((NC, N, D), jnp.float32),
    mesh=_mesh,
    scratch_types=[
        pltpu.VMEM((CHUNK,), jnp.int32),
        pltpu.VMEM((CHUNK,), jnp.int32),
        pltpu.VMEM((CHUNK, D), jnp.float32),
        pltpu.VMEM_SHARED((ACC_ROWS, D), jnp.float32),
        pltpu.SemaphoreType.DMA,
    ],
)
def _sc_agg(h_hbm, z_hbm, src_hbm, dst_hbm, out_hbm, src_v, dst_v, rows_v,
            acc_sh, sem):
    c = lax.axis_index("c")
    s = lax.axis_index("s")
    wid = s * NC + c

    def init_from(ref):
        @pl.when(s < NS - 1)
        def _():
            pltpu.sync_copy(ref.at[pl.ds(s * RPS, RPS)],
                            acc_sh.at[pl.ds(s * RPS, RPS)])

        @pl.when(s == NS - 1)
        def _():
            pltpu.sync_copy(ref.at[pl.ds((NS - 1) * RPS, LAST_INIT)],
                            acc_sh.at[pl.ds((NS - 1) * RPS, LAST_INIT)])

    @pl.when(c == 0)
    def _():
        init_from(h_hbm)

    @pl.when(c == 1)
    def _():
        init_from(z_hbm)

    plsc.subcore_barrier()

    def chunk_body(i, carry):
        off = wid * EPW + i * CHUNK
        pltpu.sync_copy(src_hbm.at[pl.ds(off, CHUNK)], src_v)
        pltpu.sync_copy(dst_hbm.at[pl.ds(off, CHUNK)], dst_v)
        pltpu.async_copy(h_hbm.at[src_v], rows_v, sem).wait()
        pltpu.sync_copy(rows_v, acc_sh.at[dst_v], add=True)
        return carry

    lax.fori_loop(0, CPW, chunk_body, 0)
    plsc.subcore_barrier()

    @pl.when(s < NS - 1)
    def _():
        pltpu.sync_copy(acc_sh.at[pl.ds(s * RPS, RPS)],
                        out_hbm.at[c, pl.ds(s * RPS, RPS)])

    @pl.when(s == NS - 1)
    def _():
        pltpu.sync_copy(acc_sh.at[pl.ds((NS - 1) * RPS, LAST_INIT)],
                        out_hbm.at[c, pl.ds((NS - 1) * RPS, LAST_INIT)])


@functools.partial(
    pl.kernel,
    out_type=jax.ShapeDtypeStruct((NW, GAD), jnp.float32),
    mesh=_mesh,
    scratch_types=[
        pltpu.VMEM((RP * D,), jnp.float32),
        pltpu.VMEM((RP + 16,), jnp.int32),
        pltpu.VMEM((GAD,), jnp.float32),
    ],
)
def _sc_pool(hf_hbm, b_hbm, ninf_hbm, out_hbm, rows_v, bid_v, acc_v):
    c = lax.axis_index("c")
    s = lax.axis_index("s")
    wid = s * NC + c

    pltpu.sync_copy(ninf_hbm, acc_v)

    @pl.when(wid < NW - 1)
    def _():
        pltpu.sync_copy(hf_hbm.at[pl.ds(wid * RP * D, RP * D)], rows_v)
        pltpu.sync_copy(b_hbm.at[pl.ds(wid * RP, RP)], bid_v.at[pl.ds(0, RP)])

    @pl.when(wid == NW - 1)
    def _():
        pltpu.sync_copy(hf_hbm.at[pl.ds((NW - 1) * RP * D, RP_LAST * D)],
                        rows_v.at[pl.ds(0, RP_LAST * D)])
        pltpu.sync_copy(b_hbm.at[pl.ds((NW - 1) * RP, RP_LAST)],
                        bid_v.at[pl.ds(0, RP_LAST)])

    nrows = jnp.where(wid == NW - 1, RP_LAST, RP)

    def row_body(r, carry):
        g = bid_v[pl.ds(r, 16)][0]
        gbase = g * D
        rbase = r * D
        for v in range(D // 16):
            xv = rows_v[pl.ds(rbase + v * 16, 16)]
            cv = acc_v[pl.ds(gbase + v * 16, 16)]
            acc_v[pl.ds(gbase + v * 16, 16)] = jnp.maximum(cv, xv)
        return carry

    lax.fori_loop(0, nrows, row_body, 0)
    pltpu.sync_copy(acc_v, out_hbm.at[wid])


def _mlp_body(parts_ref, w1_ref, b1_ref, g1_ref, be1_ref, w2_ref, b2_ref,
              bng_ref, bnb_ref, out_ref):
    z = parts_ref[0] + parts_ref[1]
    z = jnp.dot(z, w1_ref[...], preferred_element_type=jnp.float32,
                precision=lax.Precision.HIGHEST) + b1_ref[...]
    m = jnp.mean(z, axis=0)
    v = jnp.mean((z - m) ** 2, axis=0)
    z = (z - m) / jnp.sqrt(v + 1e-5) * g1_ref[...] + be1_ref[...]
    z = jnp.maximum(z, 0.0)
    z = jnp.dot(z, w2_ref[...], preferred_element_type=jnp.float32,
                precision=lax.Precision.HIGHEST) + b2_ref[...]
    m2 = jnp.mean(z, axis=0)
    v2 = jnp.mean((z - m2) ** 2, axis=0)
    z = (z - m2) / jnp.sqrt(v2 + 1e-5) * bng_ref[...] + bnb_ref[...]
    out_ref[...] = jnp.maximum(z, 0.0)


_tc_mlp = pl.pallas_call(
    _mlp_body,
    out_shape=jax.ShapeDtypeStruct((N, D), jnp.float32),
)


def _head_body(p0_ref, p1_ref, p2_ref, p3_ref,
               w10, b10, w20, b20, w11, b11, w21, b21,
               w12, b12, w22, b22, w13, b13, w23, b23, out_ref):
    score = jnp.zeros((G, OUT), jnp.float32)
    heads = [(p0_ref, w10, b10, w20, b20),
             (p1_ref, w11, b11, w21, b21),
             (p2_ref, w12, b12, w22, b22),
             (p3_ref, w13, b13, w23, b23)]
    for pref, w1, b1, w2, b2 in heads:
        arr = pref[...]
        pooled = arr[0:G, :]
        for w in range(1, NW):
            pooled = jnp.maximum(pooled, arr[w * G:(w + 1) * G, :])
        o = jnp.dot(pooled, w1[...], preferred_element_type=jnp.float32,
                    precision=lax.Precision.HIGHEST) + b1[...]
        o = jnp.maximum(o, 0.0)
        o = jnp.dot(o, w2[...], preferred_element_type=jnp.float32,
                    precision=lax.Precision.HIGHEST) + b2[...]
        score = score + o
    out_ref[...] = score


_tc_head = pl.pallas_call(
    _head_body,
    out_shape=jax.ShapeDtypeStruct((G, OUT), jnp.float32),
)


def kernel(x, edge_index, batch, params):
    src = edge_index[0]
    dst = edge_index[1]
    npad = EPAD - E
    srcp = jnp.concatenate([src, jnp.zeros((npad,), jnp.int32)])
    dstp = jnp.concatenate([dst, jnp.full((npad,), N, jnp.int32)])
    zeros_nd = jnp.zeros((N, D), jnp.float32)
    ninf = jnp.full((GAD,), -jnp.inf, jnp.float32)

    h = x
    hiddens = [x]
    for i in range(3):
        p = params["conv%d" % i]
        parts = _sc_agg(h, zeros_nd, srcp, dstp)
        h = _tc_mlp(parts, p["W1"], p["b1"], p["g1"], p["be1"],
                    p["W2"], p["b2"], p["bn_g"], p["bn_b"])
        hiddens.append(h)

    pooled_parts = [
        _sc_pool(hh.reshape(N * D), batch, ninf).reshape(NW * G, D)
        for hh in hiddens
    ]

    hp = [params["head%d" % i] for i in range(4)]
    score = _tc_head(
        pooled_parts[0], pooled_parts[1], pooled_parts[2], pooled_parts[3],
        hp[0]["W1"], hp[0]["b1"], hp[0]["W2"], hp[0]["b2"],
        hp[1]["W1"], hp[1]["b1"], hp[1]["W2"], hp[1]["b2"],
        hp[2]["W1"], hp[2]["b1"], hp[2]["W2"], hp[2]["b2"],
        hp[3]["W1"], hp[3]["b1"], hp[3]["W2"], hp[3]["b2"],
    )
    return score
